# baseline (speedup 1.0000x reference)
import jax
import jax.numpy as jnp
from jax import lax
from jax.experimental import pallas as pl
from jax.experimental.pallas import tpu as pltpu
from jax.experimental.pallas import tpu_sc as plsc

NUM_CATEGORIES = 1000000
EMBED_DIM = 64
BATCH = 4096
HIST_LEN = 50

NC = 2
NS = 16
NW = NC * NS

B_TOTAL = BATCH * HIST_LEN
B_PER_W = B_TOTAL // NW
ROWS = 80
NBUF = 8
NBATCH = B_PER_W // ROWS
PREF = 4


def _body(idx_hbm, table_hbm, out_hbm, idx_v, rows_v,
          sem_g0, sem_g1, sem_g2, sem_g3, sem_w0, sem_w1, sem_w2, sem_w3):
    wid = lax.axis_index("s") * NC + lax.axis_index("c")
    base = wid * B_PER_W
    sems_g = (sem_g0, sem_g1, sem_g2, sem_g3)
    sems_w = (sem_w0, sem_w1, sem_w2, sem_w3)
    pltpu.sync_copy(idx_hbm.at[pl.ds(base, B_PER_W)], idx_v)

    def issue(g, b, par):
        for v16 in range(ROWS // 16):
            vec = idx_v[pl.ds(g * ROWS + v16 * 16, 16)]
            offs = [vec[j] for j in range(16)]
            for j in range(16):
                pltpu.async_copy(
                    table_hbm.at[pl.ds(offs[j], 1)],
                    rows_v.at[b].at[pl.ds(v16 * 16 + j, 1)],
                    sems_g[par],
                )

    def drain_gather(b, par):
        pltpu.make_async_copy(
            table_hbm.at[pl.ds(0, ROWS)],
            rows_v.at[b],
            sems_g[par],
        ).wait()

    def start_wb(g, b, par):
        pltpu.async_copy(
            rows_v.at[b],
            out_hbm.at[pl.ds(base + g * ROWS, ROWS)],
            sems_w[par],
        )

    def wait_wb(g, b, par):
        pltpu.make_async_copy(
            rows_v.at[b],
            out_hbm.at[pl.ds(base + g * ROWS, ROWS)],
            sems_w[par],
        ).wait()

    for g0 in range(PREF):
        issue(g0, g0, g0 % 4)

    def step(t, _):
        for p in range(NBUF):
            g = NBUF * t + p
            par = p % 4
            drain_gather(p, par)
            slot = (p + PREF) % NBUF
            @pl.when(g + PREF < NBATCH)
            def _():
                @pl.when(g + PREF - NBUF >= 0)
                def _():
                    wait_wb(g + PREF - NBUF, slot, slot % 4)
                issue(g + PREF, slot, slot % 4)
            start_wb(g, p, par)
        return ()

    lax.fori_loop(0, NBATCH // NBUF, step, (), unroll=False)

    for g in range(NBATCH - NBUF, NBATCH):
        wait_wb(g, g % NBUF, g % 4)


@jax.jit
def _gather(idx1, table):
    mesh = plsc.VectorSubcoreMesh(
        core_axis_name="c", subcore_axis_name="s", num_cores=NC,
        num_subcores=NS)
    return pl.kernel(
        _body,
        out_type=jax.ShapeDtypeStruct((B_TOTAL, EMBED_DIM), jnp.float32),
        mesh=mesh,
        scratch_types=[
            pltpu.VMEM((B_PER_W,), jnp.int32),
            pltpu.VMEM((NBUF, ROWS, EMBED_DIM), jnp.float32),
            pltpu.SemaphoreType.DMA,
            pltpu.SemaphoreType.DMA,
            pltpu.SemaphoreType.DMA,
            pltpu.SemaphoreType.DMA,
            pltpu.SemaphoreType.DMA,
            pltpu.SemaphoreType.DMA,
            pltpu.SemaphoreType.DMA,
            pltpu.SemaphoreType.DMA,
        ],
        compiler_params=pltpu.CompilerParams(use_tc_tiling_on_sc=True),
    )(idx1, table)


def kernel(inputs, table):
    return _gather(inputs.reshape(-1), table)
